# Initial kernel scaffold; baseline (speedup 1.0000x reference)
import jax
import jax.numpy as jnp
from jax.experimental import pallas as pl


def kernel(trees, indexes, W_enc, b_enc, Wc1, bc1, Wc2, bc2, Wc3, bc3, W_lat, b_lat, W_cost, b_cost):
    raise NotImplementedError("write your pallas kernel here")


# speedup vs baseline: 408.8805x; 408.8805x over previous
import jax
import jax.numpy as jnp
from jax.experimental import pallas as pl


_B, _N, _CIN = 256, 511, 318
_NP1 = _N + 1


def _gather_nodes(x, idx_row):
    c = x.shape[0]
    bidx = jnp.broadcast_to(idx_row[None, :], (c, _NP1))
    lo = bidx & 127
    hi = bidx >> 7
    out = jnp.zeros((c, _NP1), x.dtype)
    for j in range(4):
        g = jnp.take_along_axis(x[:, j * 128:(j + 1) * 128], lo, axis=1,
                                mode="promise_in_bounds")
        out = jnp.where(hi == j, g, out)
    return out


def _layer(x, idx, w_cat, bias, mask):
    g = jnp.concatenate([_gather_nodes(x, idx[k]) for k in range(3)],
                        axis=0)
    out = jnp.dot(w_cat, g, preferred_element_type=jnp.float32) + bias
    out = out * mask
    n = out.shape[0] * out.shape[1]
    m = jnp.mean(out, keepdims=True)
    d = out - m
    var = jnp.sum(d * d, keepdims=True) / (n - 1)
    return jnp.maximum(d / (jnp.sqrt(var) + 1e-5), 0.0)


def _fused(idx_ref, trees_ref, wenc_ref, benc_ref, w1_ref, b1_ref,
           w2_ref, b2_ref, w3_ref, b3_ref, whead_ref, bhead_ref,
           lat_ref, cost_ref):
    idx = idx_ref[0]
    mask = (jax.lax.broadcasted_iota(jnp.int32, (1, _NP1), 1) >= 1).astype(
        jnp.float32)

    x = jnp.dot(wenc_ref[...], trees_ref[0],
                preferred_element_type=jnp.float32) + benc_ref[...]
    x = _layer(x, idx, w1_ref[...], b1_ref[...], mask)
    x = _layer(x, idx, w2_ref[...], b2_ref[...], mask)
    x = _layer(x, idx, w3_ref[...], b3_ref[...], mask)

    pooled = jnp.max(x, axis=1, keepdims=True)
    z = jnp.sum(pooled * whead_ref[...], axis=0, keepdims=True)
    s = jax.nn.sigmoid(z + bhead_ref[...])
    lat_ref[...] = jnp.broadcast_to(s[:, 0:1], (1, 128))[None]
    cost_ref[...] = jnp.broadcast_to(s[:, 1:2], (1, 128))[None]


def kernel(trees, indexes, W_enc, b_enc, Wc1, bc1, Wc2, bc2, Wc3, bc3,
           W_lat, b_lat, W_cost, b_cost):
    B = trees.shape[0]
    idx3 = indexes[:, :, 0].astype(jnp.int32).reshape(B, _N, 3)
    idx3 = jnp.transpose(idx3, (0, 2, 1))
    idx_sh = jnp.concatenate(
        [jnp.zeros((B, 3, 1), jnp.int32), idx3], axis=2)

    def cat(w):
        o, c, _ = w.shape
        return jnp.moveaxis(w, 2, 1).reshape(o, 3 * c)

    wenc_t = W_enc.T
    w1, w2, w3 = cat(Wc1), cat(Wc2), cat(Wc3)
    whead = jnp.concatenate([W_lat, W_cost], axis=1)
    bhead = jnp.concatenate([b_lat, b_cost])[None, :]

    def whole(a):
        return pl.BlockSpec(a.shape, lambda b: (0,) * a.ndim)

    consts = [wenc_t, b_enc[:, None], w1, bc1[:, None], w2, bc2[:, None],
              w3, bc3[:, None], whead, bhead]
    lat, cost = pl.pallas_call(
        _fused,
        grid=(B,),
        in_specs=[
            pl.BlockSpec((1, 3, _NP1), lambda b: (b, 0, 0)),
            pl.BlockSpec((1, _CIN, _NP1), lambda b: (b, 0, 0)),
            *[whole(a) for a in consts],
        ],
        out_specs=[
            pl.BlockSpec((1, 1, 128), lambda b: (b, 0, 0)),
            pl.BlockSpec((1, 1, 128), lambda b: (b, 0, 0)),
        ],
        out_shape=[
            jax.ShapeDtypeStruct((B, 1, 128), jnp.float32),
            jax.ShapeDtypeStruct((B, 1, 128), jnp.float32),
        ],
    )(idx_sh, trees, *consts)
    return lat[:, 0, :1], cost[:, 0, :1]


# speedup vs baseline: 787.1267x; 1.9251x over previous
import jax
import jax.numpy as jnp
from jax.experimental import pallas as pl


_B, _N, _CIN = 256, 511, 318
_NP1 = _N + 1


def _ln_relu(out, o):
    n = o * _NP1
    m = jnp.mean(out, keepdims=True)
    d = out - m
    var = jnp.sum(d * d, keepdims=True) / (n - 1)
    return jnp.maximum(d / (jnp.sqrt(var) + 1e-5), 0.0)


def _fused(idx_ref, trees_ref, wenc_ref, benc_ref, w1_ref, b1_ref,
           w2_ref, b2_ref, w3_ref, b3_ref, whead_ref, bhead_ref,
           lat_ref, cost_ref):
    f32, bf16 = jnp.float32, jnp.bfloat16
    idx = idx_ref[0]
    iota_sub = jax.lax.broadcasted_iota(jnp.int32, (_NP1, _NP1), 0)
    s_mats = [(iota_sub == idx[k][None, :]).astype(bf16) for k in range(3)]
    lane = jax.lax.broadcasted_iota(jnp.int32, (1, _NP1), 1)
    maskrow = (lane >= 1).astype(bf16)
    ones_row = jnp.ones((1, _NP1), bf16)

    def mm(a, b, out_dtype):
        return jnp.dot(a, b, preferred_element_type=out_dtype)

    x = mm(wenc_ref[...], trees_ref[0], f32) + mm(benc_ref[...], ones_row,
                                                  f32)
    xb = x.astype(bf16)

    w1 = w1_ref[...]
    acc = mm(b1_ref[...], maskrow, f32)
    for k in range(3):
        g = mm(xb, s_mats[k], f32).astype(bf16)
        acc = acc + mm(w1[k * 512:(k + 1) * 512], g, f32)
    xb = _ln_relu(acc, 512).astype(bf16)

    for w_ref, b_ref, o in ((w2_ref, b2_ref, 256), (w3_ref, b3_ref, 128)):
        y = mm(w_ref[...], xb, f32).astype(bf16)
        acc = mm(b_ref[...], maskrow, f32)
        for k in range(3):
            acc = acc + mm(y[k * o:(k + 1) * o], s_mats[k], f32)
        x = _ln_relu(acc, o)
        xb = x.astype(bf16)

    pooled = jnp.max(x, axis=1, keepdims=True)
    z = jnp.sum(pooled * whead_ref[...], axis=0, keepdims=True)
    s = jax.nn.sigmoid(z + bhead_ref[...])
    lat_ref[...] = jnp.broadcast_to(s[:, 0:1], (1, 128))[None]
    cost_ref[...] = jnp.broadcast_to(s[:, 1:2], (1, 128))[None]


def kernel(trees, indexes, W_enc, b_enc, Wc1, bc1, Wc2, bc2, Wc3, bc3,
           W_lat, b_lat, W_cost, b_cost):
    B = trees.shape[0]
    idx3 = indexes[:, :, 0].astype(jnp.int32).reshape(B, _N, 3)
    idx3 = jnp.transpose(idx3, (0, 2, 1))
    idx_sh = jnp.concatenate(
        [jnp.full((B, 3, 1), -1, jnp.int32), idx3], axis=2)

    def cat(w):
        o, c, _ = w.shape
        return jnp.moveaxis(w, 2, 0).reshape(3 * o, c).astype(jnp.bfloat16)

    wenc_t = W_enc.T.astype(jnp.bfloat16)
    trees_b = trees.astype(jnp.bfloat16)
    w1, w2, w3 = cat(Wc1), cat(Wc2), cat(Wc3)
    whead = jnp.concatenate([W_lat, W_cost], axis=1)
    bhead = jnp.concatenate([b_lat, b_cost])[None, :]
    bb = jnp.bfloat16

    def whole(a):
        return pl.BlockSpec(a.shape, lambda b: (0,) * a.ndim)

    consts = [wenc_t, b_enc[:, None].astype(bb), w1, bc1[:, None].astype(bb),
              w2, bc2[:, None].astype(bb), w3, bc3[:, None].astype(bb),
              whead, bhead]
    lat, cost = pl.pallas_call(
        _fused,
        grid=(B,),
        in_specs=[
            pl.BlockSpec((1, 3, _NP1), lambda b: (b, 0, 0)),
            pl.BlockSpec((1, _CIN, _NP1), lambda b: (b, 0, 0)),
            *[whole(a) for a in consts],
        ],
        out_specs=[
            pl.BlockSpec((1, 1, 128), lambda b: (b, 0, 0)),
            pl.BlockSpec((1, 1, 128), lambda b: (b, 0, 0)),
        ],
        out_shape=[
            jax.ShapeDtypeStruct((B, 1, 128), jnp.float32),
            jax.ShapeDtypeStruct((B, 1, 128), jnp.float32),
        ],
    )(idx_sh, trees_b, *consts)
    return lat[:, 0, :1], cost[:, 0, :1]


# speedup vs baseline: 1428.3793x; 1.8147x over previous
import jax
import jax.numpy as jnp
from jax.experimental import pallas as pl


_B, _N, _CIN = 256, 511, 318
_NP1 = _N + 1
_S = 4


def _ln_relu(out, o):
    n = o * _NP1
    s1 = jnp.sum(out, keepdims=True)
    s2 = jnp.sum(out * out, keepdims=True)
    m = s1 / n
    var = (s2 - m * s1) / (n - 1)
    return jnp.maximum((out - m) / (jnp.sqrt(var) + 1e-5), 0.0)


def _one_sample(idx, trees, wenc_ref, benc_ref, w1_ref, b1_ref,
                w2_ref, b2_ref, w3_ref, b3_ref, whead_ref, bhead_ref):
    f32, bf16 = jnp.float32, jnp.bfloat16
    iota_sub = jax.lax.broadcasted_iota(jnp.int32, (_NP1, _NP1), 0)
    s_mats = [(iota_sub == idx[k][None, :]).astype(bf16) for k in range(3)]

    def mm(a, b, out_dtype):
        return jnp.dot(a, b, preferred_element_type=out_dtype)

    x = mm(wenc_ref[...], trees, f32) + benc_ref[...]
    xb = x.astype(bf16)

    w1 = w1_ref[...]
    acc = b1_ref[...]
    for k in range(3):
        g = mm(xb, s_mats[k], f32).astype(bf16)
        acc = acc + mm(w1[k * 512:(k + 1) * 512], g, f32)
    xb = _ln_relu(acc, 512).astype(bf16)

    for w_ref, b_ref, o in ((w2_ref, b2_ref, 256), (w3_ref, b3_ref, 128)):
        y = mm(w_ref[...], xb, f32).astype(bf16)
        acc = b_ref[...]
        for k in range(3):
            acc = acc + mm(y[k * o:(k + 1) * o], s_mats[k], f32)
        x = _ln_relu(acc, o)
        xb = x.astype(bf16)

    pooled = jnp.max(x, axis=1, keepdims=True)
    z = jnp.sum(pooled * whead_ref[...], axis=0, keepdims=True)
    return jax.nn.sigmoid(z + bhead_ref[...])


def _fused(idx_ref, trees_ref, wenc_ref, benc_ref, w1_ref, b1_ref,
           w2_ref, b2_ref, w3_ref, b3_ref, whead_ref, bhead_ref,
           lat_ref, cost_ref):
    for s in range(_S):
        sig = _one_sample(idx_ref[s], trees_ref[s], wenc_ref, benc_ref,
                          w1_ref, b1_ref, w2_ref, b2_ref, w3_ref, b3_ref,
                          whead_ref, bhead_ref)
        lat_ref[s] = jnp.broadcast_to(sig[:, 0:1], (1, 128))
        cost_ref[s] = jnp.broadcast_to(sig[:, 1:2], (1, 128))


def kernel(trees, indexes, W_enc, b_enc, Wc1, bc1, Wc2, bc2, Wc3, bc3,
           W_lat, b_lat, W_cost, b_cost):
    B = trees.shape[0]
    idx3 = indexes[:, :, 0].astype(jnp.int32).reshape(B, _N, 3)
    idx3 = jnp.transpose(idx3, (0, 2, 1))
    idx_sh = jnp.concatenate(
        [jnp.full((B, 3, 1), -1, jnp.int32), idx3], axis=2)

    def cat(w):
        o, c, _ = w.shape
        return jnp.moveaxis(w, 2, 0).reshape(3 * o, c).astype(jnp.bfloat16)

    wenc_t = W_enc.T.astype(jnp.bfloat16)
    trees_b = trees.astype(jnp.bfloat16)
    w1, w2, w3 = cat(Wc1), cat(Wc2), cat(Wc3)
    whead = jnp.concatenate([W_lat, W_cost], axis=1)
    bhead = jnp.concatenate([b_lat, b_cost])[None, :]

    maskrow = (jnp.arange(_NP1) >= 1).astype(jnp.float32)[None, :]
    benc_map = jnp.broadcast_to(b_enc[:, None], (109, _NP1))
    b1_map = bc1[:, None] * maskrow
    b2_map = bc2[:, None] * maskrow
    b3_map = bc3[:, None] * maskrow

    def whole(a):
        return pl.BlockSpec(a.shape, lambda b: (0,) * a.ndim)

    consts = [wenc_t, benc_map, w1, b1_map, w2, b2_map, w3, b3_map,
              whead, bhead]
    lat, cost = pl.pallas_call(
        _fused,
        grid=(B // _S,),
        in_specs=[
            pl.BlockSpec((_S, 3, _NP1), lambda b: (b, 0, 0)),
            pl.BlockSpec((_S, _CIN, _NP1), lambda b: (b, 0, 0)),
            *[whole(a) for a in consts],
        ],
        out_specs=[
            pl.BlockSpec((_S, 1, 128), lambda b: (b, 0, 0)),
            pl.BlockSpec((_S, 1, 128), lambda b: (b, 0, 0)),
        ],
        out_shape=[
            jax.ShapeDtypeStruct((B, 1, 128), jnp.float32),
            jax.ShapeDtypeStruct((B, 1, 128), jnp.float32),
        ],
    )(idx_sh, trees_b, *consts)
    return lat[:, 0, :1], cost[:, 0, :1]
